# baseline (speedup 1.0000x reference)
import functools

import jax
import jax.numpy as jnp
from jax import lax
from jax.experimental import pallas as pl
from jax.experimental.pallas import tpu as pltpu
from jax.experimental.pallas import tpu_sc as plsc

VOCAB = 256
LANES = 16


def kernel(sents, lengths, forgetting_factor):
    B, S, W = sents.shape
    N = B * S
    NC, NS = 2, 16
    NW = NC * NS
    tok_per_w = N // NW
    G = tok_per_w // LANES
    s_per_w = S // (NW // B) if NW >= B else S * (B // NW)

    alpha_vec = jnp.broadcast_to(
        forgetting_factor.astype(jnp.float32), (LANES,))

    mesh = plsc.VectorSubcoreMesh(core_axis_name="c", subcore_axis_name="s")

    @functools.partial(
        pl.kernel,
        mesh=mesh,
        out_type=jax.ShapeDtypeStruct((B, S, VOCAB), jnp.float32),
        compiler_params=pltpu.CompilerParams(
            needs_layout_passes=False, skip_device_barrier=True),
        scratch_types=[
            pltpu.VMEM((tok_per_w, W), jnp.int32),
            pltpu.VMEM((LANES,), jnp.float32),
            pltpu.VMEM((LANES, VOCAB), jnp.float32),
            pltpu.VMEM((LANES, VOCAB), jnp.float32),
            pltpu.SemaphoreType.DMA,
            pltpu.SemaphoreType.DMA,
        ],
    )
    def fofe(sents_hbm, alpha_hbm, out_hbm, chars_v, alpha_v, acc_a, acc_b,
             sem_a, sem_b):
        wid = lax.axis_index("s") * NC + lax.axis_index("c")
        batch = wid // (NW // B)
        s_base = (wid % (NW // B)) * tok_per_w

        pltpu.sync_copy(sents_hbm.at[batch, pl.ds(s_base, tok_per_w)], chars_v)
        pltpu.sync_copy(alpha_hbm, alpha_v)

        alpha = alpha_v[...]
        lane = lax.iota(jnp.int32, 16)
        zeros16 = jnp.zeros((LANES,), jnp.float32)
        ones16 = jnp.ones((LANES,), jnp.float32)

        bufs = (acc_a, acc_b)
        sems = (sem_a, sem_b)

        def zero_body(k, _):
            for r in range(LANES):
                acc_a[r, pl.ds(k * LANES, LANES)] = zeros16
                acc_b[r, pl.ds(k * LANES, LANES)] = zeros16
            return _
        lax.fori_loop(0, VOCAB // LANES, zero_body, None)

        dma = [None, None]
        prev_chars = [None, None]
        for g in range(G):
            b = g & 1
            acc = bufs[b]
            if dma[b] is not None:
                dma[b].wait()
                for c in prev_chars[b]:
                    plsc.store_scatter(acc, [lane, c], zeros16, mask=c != 0)

            tok = lane + g * LANES
            cs = [plsc.load_gather(chars_v, [tok, jnp.full((LANES,), w,
                                                           jnp.int32)])
                  for w in range(W - 1, -1, -1)]
            p = ones16
            for c in cs:
                m = c != 0
                plsc.addupdate_scatter(acc, [lane, c], p, mask=m)
                p = jnp.where(m, p * alpha, p)
            prev_chars[b] = cs

            dma[b] = pltpu.async_copy(
                acc, out_hbm.at[batch, pl.ds(s_base + g * LANES, LANES)],
                sems[b])

        dma[0].wait()
        dma[1].wait()

    out = fofe(sents, alpha_vec)
    return (out, lengths)


# speedup vs baseline: 1.1015x; 1.1015x over previous
import functools

import jax
import jax.numpy as jnp
from jax import lax
from jax.experimental import pallas as pl
from jax.experimental.pallas import tpu as pltpu
from jax.experimental.pallas import tpu_sc as plsc

VOCAB = 256
LANES = 16


def kernel(sents, lengths, forgetting_factor):
    B, S, W = sents.shape
    N = B * S
    NC, NS = 2, 16
    NW = NC * NS
    tok_per_w = N // NW
    G = tok_per_w // LANES
    sents_t = jnp.transpose(sents, (2, 0, 1))
    alpha_1 = forgetting_factor.astype(jnp.float32).reshape(1)

    mesh = plsc.VectorSubcoreMesh(core_axis_name="c", subcore_axis_name="s")

    @functools.partial(
        pl.kernel,
        mesh=mesh,
        out_type=jax.ShapeDtypeStruct((B, S, VOCAB), jnp.float32),
        compiler_params=pltpu.CompilerParams(needs_layout_passes=False),
        scratch_types=[
            pltpu.VMEM((W, tok_per_w), jnp.int32),
            pltpu.VMEM((LANES,), jnp.float32),
            pltpu.VMEM((LANES, VOCAB), jnp.float32),
            pltpu.VMEM((LANES, VOCAB), jnp.float32),
            pltpu.SemaphoreType.DMA,
            pltpu.SemaphoreType.DMA,
        ],
    )
    def fofe(sents_hbm, alpha_hbm, out_hbm, chars_v, alpha_v, acc_a, acc_b,
             sem_a, sem_b):
        wid = lax.axis_index("s") * NC + lax.axis_index("c")
        batch = wid // (NW // B)
        s_base = (wid % (NW // B)) * tok_per_w

        pltpu.sync_copy(
            sents_hbm.at[:, batch, pl.ds(s_base, tok_per_w)], chars_v)
        pltpu.sync_copy(alpha_hbm, alpha_v.at[pl.ds(0, 1)])

        alpha = jnp.full((LANES,), alpha_v[...][0], jnp.float32)
        lane = lax.iota(jnp.int32, 16)
        zeros16 = jnp.zeros((LANES,), jnp.float32)
        ones16 = jnp.ones((LANES,), jnp.float32)

        bufs = (acc_a, acc_b)
        sems = (sem_a, sem_b)

        def zero_body(k, _):
            for r in range(LANES):
                acc_a[r, pl.ds(k * LANES, LANES)] = zeros16
                acc_b[r, pl.ds(k * LANES, LANES)] = zeros16
            return _
        lax.fori_loop(0, VOCAB // LANES, zero_body, None)

        dma = [None, None]
        prev_chars = [None, None]
        for g in range(G):
            b = g & 1
            acc = bufs[b]
            if dma[b] is not None:
                dma[b].wait()
                for c in prev_chars[b]:
                    plsc.store_scatter(acc, [lane, c], zeros16, mask=c != 0)

            tok = lane + g * LANES
            cs = [plsc.load_gather(chars_v, [jnp.full((LANES,), w, jnp.int32),
                                             tok])
                  for w in range(W - 1, -1, -1)]
            p = ones16
            for c in cs:
                m = c != 0
                plsc.addupdate_scatter(acc, [lane, c], p, mask=m)
                p = jnp.where(m, p * alpha, p)
            prev_chars[b] = cs

            dma[b] = pltpu.async_copy(
                acc, out_hbm.at[batch, pl.ds(s_base + g * LANES, LANES)],
                sems[b])

        dma[0].wait()
        dma[1].wait()

    out = fofe(sents_t, alpha_1)
    return (out, lengths)


# speedup vs baseline: 1.1103x; 1.0079x over previous
import functools

import jax
import jax.numpy as jnp
from jax import lax
from jax.experimental import pallas as pl
from jax.experimental.pallas import tpu as pltpu
from jax.experimental.pallas import tpu_sc as plsc

VOCAB = 256
LANES = 16


def kernel(sents, lengths, forgetting_factor):
    B, S, W = sents.shape
    N = B * S
    NC, NS = 2, 16
    NW = NC * NS
    tok_per_w = N // NW
    G = tok_per_w // LANES

    sents_t = jnp.transpose(sents, (2, 0, 1))
    alpha_1 = forgetting_factor.astype(jnp.float32).reshape(1)

    mesh = plsc.VectorSubcoreMesh(core_axis_name="c", subcore_axis_name="s")

    @functools.partial(
        pl.kernel,
        mesh=mesh,
        out_type=jax.ShapeDtypeStruct((B, S, VOCAB), jnp.float32),
        compiler_params=pltpu.CompilerParams(needs_layout_passes=False),
        scratch_types=[
            pltpu.VMEM((W, tok_per_w), jnp.int32),
            pltpu.VMEM((LANES,), jnp.float32),
            pltpu.VMEM((2, LANES, VOCAB), jnp.float32),
            pltpu.SemaphoreType.DMA,
        ],
    )
    def fofe(sents_hbm, alpha_hbm, out_hbm, chars_v, alpha_v, acc_v, sem):
        wid = lax.axis_index("s") * NC + lax.axis_index("c")
        batch = wid // (NW // B)
        s_base = (wid % (NW // B)) * tok_per_w

        pltpu.sync_copy(
            sents_hbm.at[:, batch, pl.ds(s_base, tok_per_w)], chars_v)
        pltpu.sync_copy(alpha_hbm, alpha_v.at[pl.ds(0, 1)])

        alpha = jnp.full((LANES,), alpha_v[...][0], jnp.float32)
        lane = lax.iota(jnp.int32, 16)
        zeros16 = jnp.zeros((LANES,), jnp.float32)
        ones16 = jnp.ones((LANES,), jnp.float32)

        def zero_body(k, _):
            for p in range(2):
                for r in range(LANES):
                    acc_v[p, r, pl.ds(k * LANES, LANES)] = zeros16
            return _
        lax.fori_loop(0, VOCAB // LANES, zero_body, None)

        def one_group(g, undo, par):
            tok = lane + g * LANES
            p = ones16
            for w in range(W - 1, -1, -1):
                wv = jnp.full((LANES,), w, jnp.int32)
                c = plsc.load_gather(chars_v, [wv, tok])
                m = c != 0
                if undo:
                    plsc.store_scatter(acc_v, [par, lane, c], zeros16, mask=m)
                else:
                    plsc.addupdate_scatter(acc_v, [par, lane, c], p, mask=m)
                    p = jnp.where(m, p * alpha, p)

        def group_body(g, _):
            b = jnp.bitwise_and(g, 1)
            par = jnp.full((LANES,), 0, jnp.int32) + b

            @pl.when(g >= 2)
            def _wait_and_undo():
                pltpu.make_async_copy(
                    acc_v.at[b],
                    out_hbm.at[batch, pl.ds(s_base, LANES)], sem).wait()
                one_group(g - 2, True, par)

            one_group(g, False, par)
            pltpu.async_copy(
                acc_v.at[b],
                out_hbm.at[batch, pl.ds(s_base + g * LANES, LANES)], sem)
            return _

        lax.fori_loop(0, G, group_body, None)

        for _ in range(2):
            pltpu.make_async_copy(
                acc_v.at[0],
                out_hbm.at[batch, pl.ds(s_base, LANES)], sem).wait()

    out = fofe(sents_t, alpha_1)
    return (out, lengths)


# speedup vs baseline: 1.2519x; 1.1276x over previous
import functools

import jax
import jax.numpy as jnp
from jax import lax
from jax.experimental import pallas as pl
from jax.experimental.pallas import tpu as pltpu
from jax.experimental.pallas import tpu_sc as plsc

VOCAB = 256
LANES = 16


def kernel(sents, lengths, forgetting_factor):
    B, S, W = sents.shape
    N = B * S
    NC, NS = 2, 16
    NW = NC * NS
    tok_per_w = N // NW
    G = tok_per_w // LANES

    sents_t = jnp.transpose(sents, (2, 0, 1))
    alpha_1 = forgetting_factor.astype(jnp.float32).reshape(1)

    mesh = plsc.VectorSubcoreMesh(core_axis_name="c", subcore_axis_name="s")

    @functools.partial(
        pl.kernel,
        mesh=mesh,
        out_type=jax.ShapeDtypeStruct((B, S, VOCAB), jnp.float32),
        compiler_params=pltpu.CompilerParams(needs_layout_passes=False),
        scratch_types=[
            pltpu.VMEM((W, tok_per_w), jnp.int32),
            pltpu.VMEM((LANES,), jnp.float32),
            pltpu.VMEM((4, LANES, VOCAB), jnp.float32),
            pltpu.SemaphoreType.DMA,
        ],
    )
    def fofe(sents_hbm, alpha_hbm, out_hbm, chars_v, alpha_v, acc_v, sem):
        wid = lax.axis_index("s") * NC + lax.axis_index("c")
        batch = wid // (NW // B)
        s_base = (wid % (NW // B)) * tok_per_w

        pltpu.sync_copy(
            sents_hbm.at[:, batch, pl.ds(s_base, tok_per_w)], chars_v)
        pltpu.sync_copy(alpha_hbm, alpha_v.at[pl.ds(0, 1)])

        alpha = jnp.full((LANES,), alpha_v[...][0], jnp.float32)
        lane = lax.iota(jnp.int32, 16)
        zeros16 = jnp.zeros((LANES,), jnp.float32)
        ones16 = jnp.ones((LANES,), jnp.float32)

        def zero_body(k, _):
            for p in range(4):
                for r in range(LANES):
                    acc_v[p, r, pl.ds(k * LANES, LANES)] = zeros16
            return _
        lax.fori_loop(0, VOCAB // LANES, zero_body, None)

        def gather_chars(g):
            tok = lane + g * LANES
            return [plsc.load_gather(
                        chars_v, [jnp.full((LANES,), w, jnp.int32), tok])
                    for w in range(W - 1, -1, -1)]

        def group_body(g, _):
            b = jnp.bitwise_and(g, 3)
            par = jnp.full((LANES,), 0, jnp.int32) + b

            @pl.when(g >= 4)
            def _wait_and_undo():
                pltpu.make_async_copy(
                    acc_v.at[b],
                    out_hbm.at[batch, pl.ds(s_base, LANES)], sem).wait()
                for c in gather_chars(g - 4):
                    plsc.store_scatter(acc_v, [par, lane, c], zeros16)

            cs = gather_chars(g)
            p = ones16
            for c in cs:
                m = c != 0
                plsc.addupdate_scatter(acc_v, [par, lane, c], p, mask=m)
                p = jnp.where(m, p * alpha, p)

            pltpu.async_copy(
                acc_v.at[b],
                out_hbm.at[batch, pl.ds(s_base + g * LANES, LANES)], sem)
            return _

        lax.fori_loop(0, G, group_body, None)

        for _ in range(4):
            pltpu.make_async_copy(
                acc_v.at[0],
                out_hbm.at[batch, pl.ds(s_base, LANES)], sem).wait()

    out = fofe(sents_t, alpha_1)
    return (out, lengths)


# speedup vs baseline: 1.2639x; 1.0095x over previous
import functools

import jax
import jax.numpy as jnp
from jax import lax
from jax.experimental import pallas as pl
from jax.experimental.pallas import tpu as pltpu
from jax.experimental.pallas import tpu_sc as plsc

VOCAB = 256
LANES = 16


def kernel(sents, lengths, forgetting_factor):
    B, S, W = sents.shape
    N = B * S
    NC, NS = 2, 16
    NW = NC * NS
    tok_per_w = N // NW
    G = tok_per_w // LANES

    sents_t = jnp.transpose(sents, (2, 0, 1))
    alpha_1 = forgetting_factor.astype(jnp.float32).reshape(1)

    mesh = plsc.VectorSubcoreMesh(core_axis_name="c", subcore_axis_name="s")

    @functools.partial(
        pl.kernel,
        mesh=mesh,
        out_type=(jax.ShapeDtypeStruct((B, S, VOCAB), jnp.float32),
                  jax.ShapeDtypeStruct((B,), jnp.int32)),
        compiler_params=pltpu.CompilerParams(needs_layout_passes=False),
        scratch_types=[
            pltpu.VMEM((W, tok_per_w), jnp.int32),
            pltpu.VMEM((LANES,), jnp.float32),
            pltpu.VMEM((B,), jnp.int32),
            pltpu.VMEM((4, LANES, VOCAB), jnp.float32),
            pltpu.SemaphoreType.DMA,
        ],
    )
    def fofe(sents_hbm, alpha_hbm, len_hbm, out_hbm, len_out_hbm,
             chars_v, alpha_v, len_v, acc_v, sem):
        wid = lax.axis_index("s") * NC + lax.axis_index("c")
        batch = wid // (NW // B)
        s_base = (wid % (NW // B)) * tok_per_w

        pltpu.sync_copy(
            sents_hbm.at[:, batch, pl.ds(s_base, tok_per_w)], chars_v)
        pltpu.sync_copy(alpha_hbm, alpha_v.at[pl.ds(0, 1)])

        @pl.when(wid == 0)
        def _pass_lengths():
            pltpu.sync_copy(len_hbm, len_v)
            pltpu.sync_copy(len_v, len_out_hbm)

        alpha = jnp.full((LANES,), alpha_v[...][0], jnp.float32)
        lane = lax.iota(jnp.int32, 16)
        zeros16 = jnp.zeros((LANES,), jnp.float32)
        ones16 = jnp.ones((LANES,), jnp.float32)

        def zero_body(k, _):
            for p in range(4):
                for r in range(LANES):
                    acc_v[p, r, pl.ds(k * LANES, LANES)] = zeros16
            return _
        lax.fori_loop(0, VOCAB // LANES, zero_body, None)

        def gather_chars(g):
            tok = lane + g * LANES
            return [plsc.load_gather(
                        chars_v, [jnp.full((LANES,), w, jnp.int32), tok])
                    for w in range(W - 1, -1, -1)]

        def group_body(g, _):
            b = jnp.bitwise_and(g, 3)
            par = jnp.full((LANES,), 0, jnp.int32) + b

            @pl.when(g >= 4)
            def _wait_and_undo():
                pltpu.make_async_copy(
                    acc_v.at[b],
                    out_hbm.at[batch, pl.ds(s_base, LANES)], sem).wait()
                for c in gather_chars(g - 4):
                    plsc.store_scatter(acc_v, [par, lane, c], zeros16)

            cs = gather_chars(g)
            p = ones16
            for c in cs:
                m = c != 0
                plsc.addupdate_scatter(acc_v, [par, lane, c], p, mask=m)
                p = jnp.where(m, p * alpha, p)

            pltpu.async_copy(
                acc_v.at[b],
                out_hbm.at[batch, pl.ds(s_base + g * LANES, LANES)], sem)
            return _

        lax.fori_loop(0, G, group_body, None)

        for _ in range(4):
            pltpu.make_async_copy(
                acc_v.at[0],
                out_hbm.at[batch, pl.ds(s_base, LANES)], sem).wait()

    out, lengths_out = fofe(sents_t, alpha_1, lengths)
    return (out, lengths_out)
